# baseline (speedup 1.0000x reference)
import functools

import jax
import jax.numpy as jnp
from jax import lax
from jax.experimental import pallas as pl
from jax.experimental.pallas import tpu as pltpu
from jax.experimental.pallas import tpu_sc as plsc

N_NODES = 100000
N_EDGES = 1600000
G = 128
TILES = 16
NG = N_EDGES // G
GP_T = NG // TILES
REM = NG - GP_T * TILES
ROWS_T = N_NODES // TILES
ZROWS = 125

CHA = 4
FULL_A = GP_T // CHA
FULL_B = 780
TAIL_BASE = 780


def _common(refs_c):
    c = lax.axis_index("c")
    s = lax.axis_index("s")
    return c, s, s * ROWS_T, c * 16, s * GP_T + jnp.minimum(s, REM), \
        GP_T + jnp.where(s < REM, 1, 0) - TAIL_BASE


def _zero_and_out(acc, zbuf, out_h, r0, coff, when):
    if when == "zero":
        @pl.loop(0, ZROWS)
        def _zb(i):
            zbuf[i, :] = jnp.zeros((16,), jnp.float32)

        @pl.loop(0, ROWS_T // ZROWS)
        def _z(kk):
            pltpu.sync_copy(zbuf, acc.at[pl.ds(r0 + kk * ZROWS, ZROWS)])

        plsc.subcore_barrier()
    else:
        plsc.subcore_barrier()
        pltpu.sync_copy(acc.at[pl.ds(r0, ROWS_T)],
                        out_h.at[pl.ds(r0, ROWS_T), pl.ds(coff, 16)])


def _sc_body_a(refs):
    (feat0_h, feat1_h, src_h, dst_h, w_h, out_h,
     acc, src_v, dst_v, w_v, half_v, zbuf, sem_in, sem_g, sem_s) = refs
    CH = CHA
    c, s, r0, coff, base_g, tail = _common(refs)

    def in_descs(i, b2, b3, make):
        gb = base_g + i * CH
        op = pltpu.make_async_copy if make else pltpu.async_copy
        ds_ = []
        for j in range(CH):
            e0 = (gb + j) * G
            ds_.append(op(dst_h.at[pl.ds(e0, G)], dst_v.at[b3, j], sem_in))
            ds_.append(op(w_h.at[pl.ds(e0, G)], w_v.at[b2, j], sem_in))
            ds_.append(op(src_h.at[pl.ds(e0, G)], src_v.at[b2, j], sem_in))
        return ds_

    def fire_data(b2, b3):
        @pl.when(c == 0)
        def _f0():
            for j in range(CH):
                pltpu.async_copy(feat0_h.at[src_v.at[b2, j]],
                                 half_v.at[b3, j], sem_g)

        @pl.when(c == 1)
        def _f1():
            for j in range(CH):
                pltpu.async_copy(feat1_h.at[src_v.at[b2, j]],
                                 half_v.at[b3, j], sem_g)

    def drain_data(b2, b3):
        for j in range(CH):
            pltpu.make_async_copy(feat0_h.at[src_v.at[b2, j]],
                                  half_v.at[b3, j], sem_g).wait()

    def compute(b2, b3, nj=CH):
        for j in range(nj):
            @plsc.parallel_loop(0, G // 16, unroll=2)
            def _m(kk):
                w16 = w_v[b2, j, pl.ds(kk * 16, 16)]
                for t in range(16):
                    e = kk * 16 + t
                    half_v[b3, j, e, :] = half_v[b3, j, e, :] * w16[t]

    def fire_scatter(b3):
        for j in range(CH):
            pltpu.async_copy(half_v.at[b3, j], acc.at[dst_v.at[b3, j]],
                             sem_s.at[b3], add=True)

    def drain_scatter(b3):
        for j in range(CH):
            pltpu.make_async_copy(half_v.at[b3, j], acc.at[dst_v.at[b3, j]],
                                  sem_s.at[b3]).wait()

    _zero_and_out(acc, zbuf, out_h, r0, coff, "zero")

    for d in in_descs(0, 0, 0, make=False):
        d.wait()
    fire_data(0, 0)

    @pl.loop(0, FULL_A)
    def _chunk(i):
        b2 = lax.rem(i, 2)
        nb2 = 1 - b2
        b3 = lax.rem(i, 3)
        nb3 = lax.rem(i + 1, 3)

        @pl.when(i >= 2)
        def _dsc():
            drain_scatter(nb3)

        @pl.when(i < FULL_A - 1)
        def _pf():
            in_descs(i + 1, nb2, nb3, make=False)

        drain_data(b2, b3)
        compute(b2, b3)
        fire_scatter(b3)

        @pl.when(i < FULL_A - 1)
        def _ng():
            for d in in_descs(i + 1, nb2, nb3, make=True):
                d.wait()
            fire_data(nb2, nb3)

    drain_scatter((FULL_A - 2) % 3)
    drain_scatter((FULL_A - 1) % 3)

    @pl.loop(0, tail)
    def _tail(tg):
        e0 = (base_g + TAIL_BASE + tg) * G
        pltpu.sync_copy(dst_h.at[pl.ds(e0, G)], dst_v.at[0, 0])
        pltpu.sync_copy(w_h.at[pl.ds(e0, G)], w_v.at[0, 0])
        pltpu.sync_copy(src_h.at[pl.ds(e0, G)], src_v.at[0, 0])

        @pl.when(c == 0)
        def _t0():
            pltpu.async_copy(feat0_h.at[src_v.at[0, 0]],
                             half_v.at[0, 0], sem_g).wait()

        @pl.when(c == 1)
        def _t1():
            pltpu.async_copy(feat1_h.at[src_v.at[0, 0]],
                             half_v.at[0, 0], sem_g).wait()
        compute(0, 0, nj=1)
        pltpu.sync_copy(half_v.at[0, 0], acc.at[dst_v.at[0, 0]], add=True)

    _zero_and_out(acc, zbuf, out_h, r0, coff, "out")


E4 = N_EDGES // 4
MG = E4 // G
GP_B = MG // TILES
REM_B = MG - GP_B * TILES


def _sc_body_b(refs):
    (rw_h, dst_h, out_h, acc, dst_v, half_v, zbuf,
     sem_in, sem_g, sem_s) = refs
    c, s, r0, coff, base_g_unused, tail_unused = _common(refs)
    coff = lax.axis_index("c") * 16
    base_g = s * GP_B + jnp.minimum(s, REM_B)
    ngt = GP_B + jnp.where(s < REM_B, 1, 0)

    def dst_desc(gi, u, p, make):
        e0 = u * E4 + (base_g + gi) * G
        op = pltpu.make_async_copy if make else pltpu.async_copy
        return op(dst_h.at[pl.ds(e0, G)], dst_v.at[p, u], sem_in.at[p, u])

    def data_desc(gi, u, p, make):
        m0 = (base_g + gi) * G
        op = pltpu.make_async_copy if make else pltpu.async_copy
        return op(rw_h.at[pl.ds(m0, G), pl.ds(u * 32 + coff, 16)],
                  half_v.at[p, u], sem_g.at[p, u])

    def fire_scatter(p, u):
        pltpu.async_copy(half_v.at[p, u], acc.at[dst_v.at[p, u]],
                         sem_s.at[p, u], add=True)

    def drain_scatter(p, u):
        pltpu.make_async_copy(half_v.at[p, u], acc.at[dst_v.at[p, u]],
                              sem_s.at[p, u]).wait()

    _zero_and_out(acc, zbuf, out_h, r0, coff, "zero")

    for u in range(4):
        dst_desc(0, u, 0, make=False)
        data_desc(0, u, 0, make=False)

    @pl.loop(0, ngt)
    def _g(gi):
        p = lax.rem(gi, 2)
        np_ = 1 - p
        for u in range(4):
            @pl.when(gi >= 1)
            def _dsc():
                drain_scatter(np_, u)

            @pl.when(gi < ngt - 1)
            def _pf():
                dst_desc(gi + 1, u, np_, make=False)
                data_desc(gi + 1, u, np_, make=False)

            dst_desc(gi, u, p, make=True).wait()
            data_desc(gi, u, p, make=True).wait()
            fire_scatter(p, u)

    pl_last = lax.rem(ngt - 1, 2)
    for u in range(4):
        pltpu.make_async_copy(half_v.at[pl_last, u],
                              acc.at[dst_v.at[pl_last, u]],
                              sem_s.at[pl_last, u]).wait()

    _zero_and_out(acc, zbuf, out_h, r0, coff, "out")


def _tc_repack(rvt, w):
    BR = 128
    NB = E4 // BR

    def body(x0, x1, x2, x3, w0, w1, w2, w3, o_ref):
        for q, (x_ref, w_ref) in enumerate(
                ((x0, w0), (x1, w1), (x2, w2), (x3, w3))):
            o_ref[:, q * 32:(q + 1) * 32] = (x_ref[...] * w_ref[...]).T

    xmaps = [(lambda q: (lambda i: (0, q * NB + i)))(q) for q in range(4)]
    wmaps = [(lambda q: (lambda i: (q * NB + i,)))(q) for q in range(4)]
    return pl.pallas_call(
        body,
        grid=(NB,),
        in_specs=[pl.BlockSpec((32, BR), m) for m in xmaps]
                 + [pl.BlockSpec((BR,), m) for m in wmaps],
        out_specs=pl.BlockSpec((BR, 128), lambda i: (i, 0)),
        out_shape=jax.ShapeDtypeStruct((E4, 128), jnp.float32),
    )(rvt, rvt, rvt, rvt, w, w, w, w)


def _make_phase_a():
    mesh = plsc.VectorSubcoreMesh(core_axis_name="c", subcore_axis_name="s")

    @functools.partial(
        pl.kernel,
        out_type=jax.ShapeDtypeStruct((N_NODES, 32), jnp.float32),
        mesh=mesh,
        scratch_types=[
            pltpu.VMEM_SHARED((N_NODES, 16), jnp.float32),
            pltpu.VMEM((2, CHA, G), jnp.int32),
            pltpu.VMEM((3, CHA, G), jnp.int32),
            pltpu.VMEM((2, CHA, G), jnp.float32),
            pltpu.VMEM((3, CHA, G, 16), jnp.float32),
            pltpu.VMEM((ZROWS, 16), jnp.float32),
            pltpu.SemaphoreType.DMA,
            pltpu.SemaphoreType.DMA,
            pltpu.SemaphoreType.DMA((3,)),
        ],
        compiler_params=pltpu.CompilerParams(use_tc_tiling_on_sc=False),
    )
    def ka(*refs):
        _sc_body_a(refs)

    return ka


def _make_phase_b():
    mesh = plsc.VectorSubcoreMesh(core_axis_name="c", subcore_axis_name="s")

    @functools.partial(
        pl.kernel,
        out_type=jax.ShapeDtypeStruct((N_NODES, 32), jnp.float32),
        mesh=mesh,
        scratch_types=[
            pltpu.VMEM_SHARED((N_NODES, 16), jnp.float32),
            pltpu.VMEM((2, 4, G), jnp.int32),
            pltpu.VMEM((2, 4, G, 16), jnp.float32),
            pltpu.VMEM((ZROWS, 16), jnp.float32),
            pltpu.SemaphoreType.DMA((2, 4)),
            pltpu.SemaphoreType.DMA((2, 4)),
            pltpu.SemaphoreType.DMA((2, 4)),
        ],
        compiler_params=pltpu.CompilerParams(use_tc_tiling_on_sc=False),
    )
    def kb(*refs):
        _sc_body_b(refs)

    return kb


def _tc_matmul(a, b, wn_t, wr_t):
    BR = 2000

    def body(a_ref, b_ref, wn_ref, wr_ref, o_ref):
        o_ref[...] = (
            jnp.dot(a_ref[...], wn_ref[...], preferred_element_type=jnp.float32)
            + jnp.dot(b_ref[...], wr_ref[...], preferred_element_type=jnp.float32))

    return pl.pallas_call(
        body,
        grid=(N_NODES // BR,),
        in_specs=[pl.BlockSpec((BR, 32), lambda i: (i, 0)),
                  pl.BlockSpec((BR, 32), lambda i: (i, 0)),
                  pl.BlockSpec((32, 32), lambda i: (0, 0)),
                  pl.BlockSpec((32, 32), lambda i: (0, 0))],
        out_specs=pl.BlockSpec((BR, 32), lambda i: (i, 0)),
        out_shape=jax.ShapeDtypeStruct((N_NODES, 32), jnp.float32),
    )(a, b, wn_t, wr_t)


def kernel(feat, edge_index, review_feat, edge_weight, W_node, W_review):
    ei = edge_index.astype(jnp.int32)
    src = ei[0]
    dst = ei[1]
    w = edge_weight.reshape(-1)
    feat0 = feat[:, :16]
    feat1 = feat[:, 16:]
    rw = _tc_repack(review_feat.T, w)
    a64 = _make_phase_a()(feat0, feat1, src, dst, w)
    b64 = _make_phase_b()(rw, dst)
    return _tc_matmul(a64, b64, W_node.T, W_review.T)


# speedup vs baseline: 2.0648x; 2.0648x over previous
import functools

import jax
import jax.numpy as jnp
from jax import lax
from jax.experimental import pallas as pl
from jax.experimental.pallas import tpu as pltpu
from jax.experimental.pallas import tpu_sc as plsc

N_NODES = 100000
N_EDGES = 1600000
G = 128
TILES = 16
NG = N_EDGES // G
GP_T = NG // TILES
REM = NG - GP_T * TILES
ROWS_T = N_NODES // TILES
ZROWS = 125

CHA = 4
FULL_A = GP_T // CHA
FULL_B = 780
TAIL_BASE = 780


def _common(refs_c):
    c = lax.axis_index("c")
    s = lax.axis_index("s")
    return c, s, s * ROWS_T, c * 16, s * GP_T + jnp.minimum(s, REM), \
        GP_T + jnp.where(s < REM, 1, 0) - TAIL_BASE


def _zero_and_out(acc, zbuf, out_h, r0, coff, when):
    if when == "zero":
        @pl.loop(0, ZROWS)
        def _zb(i):
            zbuf[i, :] = jnp.zeros((16,), jnp.float32)

        @pl.loop(0, ROWS_T // ZROWS)
        def _z(kk):
            pltpu.sync_copy(zbuf, acc.at[pl.ds(r0 + kk * ZROWS, ZROWS)])

        plsc.subcore_barrier()
    else:
        plsc.subcore_barrier()
        pltpu.sync_copy(acc.at[pl.ds(r0, ROWS_T)],
                        out_h.at[pl.ds(r0, ROWS_T), pl.ds(coff, 16)])


def _sc_body_a(refs):
    (feat0_h, feat1_h, src_h, dst_h, w_h, out_h,
     acc, src_v, dst_v, w_v, half_v, zbuf, sem_in, sem_g, sem_s) = refs
    CH = CHA
    c, s, r0, coff, base_g, tail = _common(refs)

    def in_descs(i, b2, b3, make):
        gb = base_g + i * CH
        op = pltpu.make_async_copy if make else pltpu.async_copy
        ds_ = []
        for j in range(CH):
            e0 = (gb + j) * G
            ds_.append(op(dst_h.at[pl.ds(e0, G)], dst_v.at[b3, j], sem_in))
            ds_.append(op(w_h.at[pl.ds(e0, G)], w_v.at[b2, j], sem_in))
            ds_.append(op(src_h.at[pl.ds(e0, G)], src_v.at[b2, j], sem_in))
        return ds_

    def fire_data(b2, b3):
        @pl.when(c == 0)
        def _f0():
            for j in range(CH):
                pltpu.async_copy(feat0_h.at[src_v.at[b2, j]],
                                 half_v.at[b3, j], sem_g)

        @pl.when(c == 1)
        def _f1():
            for j in range(CH):
                pltpu.async_copy(feat1_h.at[src_v.at[b2, j]],
                                 half_v.at[b3, j], sem_g)

    def drain_data(b2, b3):
        for j in range(CH):
            pltpu.make_async_copy(feat0_h.at[src_v.at[b2, j]],
                                  half_v.at[b3, j], sem_g).wait()

    def compute(b2, b3, nj=CH):
        for j in range(nj):
            @plsc.parallel_loop(0, G // 16, unroll=2)
            def _m(kk):
                w16 = w_v[b2, j, pl.ds(kk * 16, 16)]
                for t in range(16):
                    e = kk * 16 + t
                    half_v[b3, j, e, :] = half_v[b3, j, e, :] * w16[t]

    def fire_scatter(b3):
        for j in range(CH):
            pltpu.async_copy(half_v.at[b3, j], acc.at[dst_v.at[b3, j]],
                             sem_s.at[b3], add=True)

    def drain_scatter(b3):
        for j in range(CH):
            pltpu.make_async_copy(half_v.at[b3, j], acc.at[dst_v.at[b3, j]],
                                  sem_s.at[b3]).wait()

    _zero_and_out(acc, zbuf, out_h, r0, coff, "zero")

    for d in in_descs(0, 0, 0, make=False):
        d.wait()
    fire_data(0, 0)

    @pl.loop(0, FULL_A)
    def _chunk(i):
        b2 = lax.rem(i, 2)
        nb2 = 1 - b2
        b3 = lax.rem(i, 3)
        nb3 = lax.rem(i + 1, 3)

        @pl.when(i >= 2)
        def _dsc():
            drain_scatter(nb3)

        @pl.when(i < FULL_A - 1)
        def _pf():
            in_descs(i + 1, nb2, nb3, make=False)

        drain_data(b2, b3)
        compute(b2, b3)
        fire_scatter(b3)

        @pl.when(i < FULL_A - 1)
        def _ng():
            for d in in_descs(i + 1, nb2, nb3, make=True):
                d.wait()
            fire_data(nb2, nb3)

    drain_scatter((FULL_A - 2) % 3)
    drain_scatter((FULL_A - 1) % 3)

    @pl.loop(0, tail)
    def _tail(tg):
        e0 = (base_g + TAIL_BASE + tg) * G
        pltpu.sync_copy(dst_h.at[pl.ds(e0, G)], dst_v.at[0, 0])
        pltpu.sync_copy(w_h.at[pl.ds(e0, G)], w_v.at[0, 0])
        pltpu.sync_copy(src_h.at[pl.ds(e0, G)], src_v.at[0, 0])

        @pl.when(c == 0)
        def _t0():
            pltpu.async_copy(feat0_h.at[src_v.at[0, 0]],
                             half_v.at[0, 0], sem_g).wait()

        @pl.when(c == 1)
        def _t1():
            pltpu.async_copy(feat1_h.at[src_v.at[0, 0]],
                             half_v.at[0, 0], sem_g).wait()
        compute(0, 0, nj=1)
        pltpu.sync_copy(half_v.at[0, 0], acc.at[dst_v.at[0, 0]], add=True)

    _zero_and_out(acc, zbuf, out_h, r0, coff, "out")


E4 = N_EDGES // 4
MG = E4 // G
GP_B = MG // TILES
REM_B = MG - GP_B * TILES


def _sc_body_b(refs):
    (rw_h, dst_h, out_h, acc, dst_v, half_v, zbuf,
     sem_in, sem_g, sem_s) = refs
    c, s, r0, coff, base_g_unused, tail_unused = _common(refs)
    coff = lax.axis_index("c") * 16
    base_g = s * GP_B + jnp.minimum(s, REM_B)
    ngt = GP_B + jnp.where(s < REM_B, 1, 0)

    def dst_desc(gi, u, p, make):
        e0 = u * E4 + (base_g + gi) * G
        op = pltpu.make_async_copy if make else pltpu.async_copy
        return op(dst_h.at[pl.ds(e0, G)], dst_v.at[p, u], sem_in.at[p, u])

    def data_desc(gi, u, p, make):
        m0 = (base_g + gi) * G
        op = pltpu.make_async_copy if make else pltpu.async_copy
        return op(rw_h.at[pl.ds(m0, G), pl.ds(u * 32 + coff, 16)],
                  half_v.at[p, u], sem_g.at[p, u])

    def fire_scatter(p, u):
        pltpu.async_copy(half_v.at[p, u], acc.at[dst_v.at[p, u]],
                         sem_s.at[p, u], add=True)

    def drain_scatter(p, u):
        pltpu.make_async_copy(half_v.at[p, u], acc.at[dst_v.at[p, u]],
                              sem_s.at[p, u]).wait()

    _zero_and_out(acc, zbuf, out_h, r0, coff, "zero")

    for u in range(4):
        dst_desc(0, u, 0, make=False)
        data_desc(0, u, 0, make=False)

    @pl.loop(0, ngt)
    def _g(gi):
        p = lax.rem(gi, 2)
        np_ = 1 - p
        for u in range(4):
            @pl.when(gi >= 1)
            def _dsc():
                drain_scatter(np_, u)

            @pl.when(gi < ngt - 1)
            def _pf():
                dst_desc(gi + 1, u, np_, make=False)
                data_desc(gi + 1, u, np_, make=False)

            dst_desc(gi, u, p, make=True).wait()
            data_desc(gi, u, p, make=True).wait()
            fire_scatter(p, u)

    pl_last = lax.rem(ngt - 1, 2)
    for u in range(4):
        pltpu.make_async_copy(half_v.at[pl_last, u],
                              acc.at[dst_v.at[pl_last, u]],
                              sem_s.at[pl_last, u]).wait()

    _zero_and_out(acc, zbuf, out_h, r0, coff, "out")


def _tc_repack(rvt, w3d):
    BR = 640
    NB = E4 // BR

    def body(x0, x1, x2, x3, w0, w1, w2, w3, o_ref):
        for q, (x_ref, w_ref) in enumerate(
                ((x0, w0), (x1, w1), (x2, w2), (x3, w3))):
            o_ref[:, q * 32:(q + 1) * 32] = (x_ref[...] * w_ref[0, 0]).T

    xmaps = [(lambda q: (lambda i: (0, q * NB + i)))(q) for q in range(4)]
    wmaps = [(lambda q: (lambda i: (q * NB + i, 0, 0)))(q) for q in range(4)]
    return pl.pallas_call(
        body,
        grid=(NB,),
        in_specs=[pl.BlockSpec((32, BR), m) for m in xmaps]
                 + [pl.BlockSpec((1, 1, BR), m) for m in wmaps],
        out_specs=pl.BlockSpec((BR, 128), lambda i: (i, 0)),
        out_shape=jax.ShapeDtypeStruct((E4, 128), jnp.float32),
    )(rvt, rvt, rvt, rvt, w3d, w3d, w3d, w3d)


def _make_phase_a():
    mesh = plsc.VectorSubcoreMesh(core_axis_name="c", subcore_axis_name="s")

    @functools.partial(
        pl.kernel,
        out_type=jax.ShapeDtypeStruct((N_NODES, 32), jnp.float32),
        mesh=mesh,
        scratch_types=[
            pltpu.VMEM_SHARED((N_NODES, 16), jnp.float32),
            pltpu.VMEM((2, CHA, G), jnp.int32),
            pltpu.VMEM((3, CHA, G), jnp.int32),
            pltpu.VMEM((2, CHA, G), jnp.float32),
            pltpu.VMEM((3, CHA, G, 16), jnp.float32),
            pltpu.VMEM((ZROWS, 16), jnp.float32),
            pltpu.SemaphoreType.DMA,
            pltpu.SemaphoreType.DMA,
            pltpu.SemaphoreType.DMA((3,)),
        ],
        compiler_params=pltpu.CompilerParams(use_tc_tiling_on_sc=False),
    )
    def ka(*refs):
        _sc_body_a(refs)

    return ka


def _make_phase_b():
    mesh = plsc.VectorSubcoreMesh(core_axis_name="c", subcore_axis_name="s")

    @functools.partial(
        pl.kernel,
        out_type=jax.ShapeDtypeStruct((N_NODES, 32), jnp.float32),
        mesh=mesh,
        scratch_types=[
            pltpu.VMEM_SHARED((N_NODES, 16), jnp.float32),
            pltpu.VMEM((2, 4, G), jnp.int32),
            pltpu.VMEM((2, 4, G, 16), jnp.float32),
            pltpu.VMEM((ZROWS, 16), jnp.float32),
            pltpu.SemaphoreType.DMA((2, 4)),
            pltpu.SemaphoreType.DMA((2, 4)),
            pltpu.SemaphoreType.DMA((2, 4)),
        ],
        compiler_params=pltpu.CompilerParams(use_tc_tiling_on_sc=False),
    )
    def kb(*refs):
        _sc_body_b(refs)

    return kb


def _tc_matmul(a, b, wn_t, wr_t):
    BR = 2000

    def body(a_ref, b_ref, wn_ref, wr_ref, o_ref):
        o_ref[...] = (
            jnp.dot(a_ref[...], wn_ref[...], preferred_element_type=jnp.float32)
            + jnp.dot(b_ref[...], wr_ref[...], preferred_element_type=jnp.float32))

    return pl.pallas_call(
        body,
        grid=(N_NODES // BR,),
        in_specs=[pl.BlockSpec((BR, 32), lambda i: (i, 0)),
                  pl.BlockSpec((BR, 32), lambda i: (i, 0)),
                  pl.BlockSpec((32, 32), lambda i: (0, 0)),
                  pl.BlockSpec((32, 32), lambda i: (0, 0))],
        out_specs=pl.BlockSpec((BR, 32), lambda i: (i, 0)),
        out_shape=jax.ShapeDtypeStruct((N_NODES, 32), jnp.float32),
    )(a, b, wn_t, wr_t)


def kernel(feat, edge_index, review_feat, edge_weight, W_node, W_review):
    ei = edge_index.astype(jnp.int32)
    src = ei[0]
    dst = ei[1]
    w = edge_weight.reshape(-1)
    feat0 = feat[:, :16]
    feat1 = feat[:, 16:]
    rw = _tc_repack(review_feat.T, w.reshape(N_EDGES // 640, 1, 640))
    a64 = _make_phase_a()(feat0, feat1, src, dst, w)
    b64 = _make_phase_b()(rw, dst)
    return _tc_matmul(a64, b64, W_node.T, W_review.T)


# speedup vs baseline: 2.3606x; 1.1433x over previous
import functools

import jax
import jax.numpy as jnp
from jax import lax
from jax.experimental import pallas as pl
from jax.experimental.pallas import tpu as pltpu
from jax.experimental.pallas import tpu_sc as plsc

N_NODES = 100000
N_EDGES = 1600000
G = 128
TILES = 16
NG = N_EDGES // G
GP_T = NG // TILES
REM = NG - GP_T * TILES
ROWS_T = N_NODES // TILES
ZROWS = 125

CHA = 4
FULL_A = GP_T // CHA
FULL_B = 780
TAIL_BASE = 780


def _common(refs_c):
    c = lax.axis_index("c")
    s = lax.axis_index("s")
    return c, s, s * ROWS_T, c * 16, s * GP_T + jnp.minimum(s, REM), \
        GP_T + jnp.where(s < REM, 1, 0) - TAIL_BASE


def _zero_and_out(acc, zbuf, out_h, r0, coff, when):
    if when == "zero":
        @pl.loop(0, ZROWS)
        def _zb(i):
            zbuf[i, :] = jnp.zeros((16,), jnp.float32)

        @pl.loop(0, ROWS_T // ZROWS)
        def _z(kk):
            pltpu.sync_copy(zbuf, acc.at[pl.ds(r0 + kk * ZROWS, ZROWS)])

        plsc.subcore_barrier()
    else:
        plsc.subcore_barrier()
        pltpu.sync_copy(acc.at[pl.ds(r0, ROWS_T)],
                        out_h.at[pl.ds(r0, ROWS_T), pl.ds(coff, 16)])


def _sc_body_a(refs):
    (feat0_h, feat1_h, src_h, dst_h, w_h, out_h,
     acc, src_v, dst_v, w_v, half_v, zbuf, sem_in, sem_g, sem_s) = refs
    CH = CHA
    c, s, r0, coff, base_g, tail = _common(refs)

    def in_descs(i, b2, b3, make):
        gb = base_g + i * CH
        op = pltpu.make_async_copy if make else pltpu.async_copy
        ds_ = []
        for j in range(CH):
            e0 = (gb + j) * G
            ds_.append(op(dst_h.at[pl.ds(e0, G)], dst_v.at[b3, j], sem_in))
            ds_.append(op(w_h.at[pl.ds(e0, G)], w_v.at[b2, j], sem_in))
            ds_.append(op(src_h.at[pl.ds(e0, G)], src_v.at[b2, j], sem_in))
        return ds_

    def fire_data(b2, b3):
        @pl.when(c == 0)
        def _f0():
            for j in range(CH):
                pltpu.async_copy(feat0_h.at[src_v.at[b2, j]],
                                 half_v.at[b3, j], sem_g)

        @pl.when(c == 1)
        def _f1():
            for j in range(CH):
                pltpu.async_copy(feat1_h.at[src_v.at[b2, j]],
                                 half_v.at[b3, j], sem_g)

    def drain_data(b2, b3):
        for j in range(CH):
            pltpu.make_async_copy(feat0_h.at[src_v.at[b2, j]],
                                  half_v.at[b3, j], sem_g).wait()

    def compute(b2, b3, nj=CH):
        for j in range(nj):
            @plsc.parallel_loop(0, G // 16, unroll=2)
            def _m(kk):
                w16 = w_v[b2, j, pl.ds(kk * 16, 16)]
                for t in range(16):
                    e = kk * 16 + t
                    half_v[b3, j, e, :] = half_v[b3, j, e, :] * w16[t]

    def fire_scatter(b3):
        for j in range(CH):
            pltpu.async_copy(half_v.at[b3, j], acc.at[dst_v.at[b3, j]],
                             sem_s.at[b3], add=True)

    def drain_scatter(b3):
        for j in range(CH):
            pltpu.make_async_copy(half_v.at[b3, j], acc.at[dst_v.at[b3, j]],
                                  sem_s.at[b3]).wait()

    _zero_and_out(acc, zbuf, out_h, r0, coff, "zero")

    for d in in_descs(0, 0, 0, make=False):
        d.wait()
    fire_data(0, 0)

    @pl.loop(0, FULL_A)
    def _chunk(i):
        b2 = lax.rem(i, 2)
        nb2 = 1 - b2
        b3 = lax.rem(i, 3)
        nb3 = lax.rem(i + 1, 3)

        @pl.when(i >= 2)
        def _dsc():
            drain_scatter(nb3)

        @pl.when(i < FULL_A - 1)
        def _pf():
            in_descs(i + 1, nb2, nb3, make=False)

        drain_data(b2, b3)
        compute(b2, b3)
        fire_scatter(b3)

        @pl.when(i < FULL_A - 1)
        def _ng():
            for d in in_descs(i + 1, nb2, nb3, make=True):
                d.wait()
            fire_data(nb2, nb3)

    drain_scatter((FULL_A - 2) % 3)
    drain_scatter((FULL_A - 1) % 3)

    @pl.loop(0, tail)
    def _tail(tg):
        e0 = (base_g + TAIL_BASE + tg) * G
        pltpu.sync_copy(dst_h.at[pl.ds(e0, G)], dst_v.at[0, 0])
        pltpu.sync_copy(w_h.at[pl.ds(e0, G)], w_v.at[0, 0])
        pltpu.sync_copy(src_h.at[pl.ds(e0, G)], src_v.at[0, 0])

        @pl.when(c == 0)
        def _t0():
            pltpu.async_copy(feat0_h.at[src_v.at[0, 0]],
                             half_v.at[0, 0], sem_g).wait()

        @pl.when(c == 1)
        def _t1():
            pltpu.async_copy(feat1_h.at[src_v.at[0, 0]],
                             half_v.at[0, 0], sem_g).wait()
        compute(0, 0, nj=1)
        pltpu.sync_copy(half_v.at[0, 0], acc.at[dst_v.at[0, 0]], add=True)

    _zero_and_out(acc, zbuf, out_h, r0, coff, "out")


E4 = N_EDGES // 4
MG = E4 // G
GP_B = MG // TILES
REM_B = MG - GP_B * TILES


def _sc_body_b(refs):
    (rw_h, dst_h, out_h, acc, dst_v, half_v, zbuf,
     sem_in, sem_g, sem_s) = refs
    c, s, r0, coff, base_g_unused, tail_unused = _common(refs)
    coff = lax.axis_index("c") * 16
    base_g = s * GP_B + jnp.minimum(s, REM_B)
    ngt = GP_B + jnp.where(s < REM_B, 1, 0)

    def dst_desc(gi, u, p, make):
        e0 = u * E4 + (base_g + gi) * G
        op = pltpu.make_async_copy if make else pltpu.async_copy
        return op(dst_h.at[pl.ds(e0, G)], dst_v.at[p, u], sem_in.at[p, u])

    def data_desc(gi, u, p, make):
        m0 = (base_g + gi) * G
        op = pltpu.make_async_copy if make else pltpu.async_copy
        return op(rw_h.at[pl.ds(m0, G), pl.ds(u * 32 + coff, 16)],
                  half_v.at[p, u], sem_g.at[p, u])

    def fire_scatter(p, u):
        pltpu.async_copy(half_v.at[p, u], acc.at[dst_v.at[p, u]],
                         sem_s.at[p, u], add=True)

    def drain_scatter(p, u):
        pltpu.make_async_copy(half_v.at[p, u], acc.at[dst_v.at[p, u]],
                              sem_s.at[p, u]).wait()

    _zero_and_out(acc, zbuf, out_h, r0, coff, "zero")

    for u in range(4):
        dst_desc(0, u, 0, make=False)
        data_desc(0, u, 0, make=False)

    @pl.loop(0, ngt)
    def _g(gi):
        p = lax.rem(gi, 2)
        np_ = 1 - p
        for u in range(4):
            @pl.when(gi >= 1)
            def _dsc():
                drain_scatter(np_, u)

            @pl.when(gi < ngt - 1)
            def _pf():
                dst_desc(gi + 1, u, np_, make=False)
                data_desc(gi + 1, u, np_, make=False)

            dst_desc(gi, u, p, make=True).wait()
            data_desc(gi, u, p, make=True).wait()
            fire_scatter(p, u)

    pl_last = lax.rem(ngt - 1, 2)
    for u in range(4):
        pltpu.make_async_copy(half_v.at[pl_last, u],
                              acc.at[dst_v.at[pl_last, u]],
                              sem_s.at[pl_last, u]).wait()

    _zero_and_out(acc, zbuf, out_h, r0, coff, "out")


def _tc_repack(rvt, w3d):
    BR = 3200
    NB = E4 // BR

    def body(x0, x1, x2, x3, w0, w1, w2, w3, o_ref):
        for q, (x_ref, w_ref) in enumerate(
                ((x0, w0), (x1, w1), (x2, w2), (x3, w3))):
            o_ref[:, q * 32:(q + 1) * 32] = (x_ref[...] * w_ref[0, 0]).T

    xmaps = [(lambda q: (lambda i: (0, q * NB + i)))(q) for q in range(4)]
    wmaps = [(lambda q: (lambda i: (q * NB + i, 0, 0)))(q) for q in range(4)]
    return pl.pallas_call(
        body,
        grid=(NB,),
        in_specs=[pl.BlockSpec((32, BR), m) for m in xmaps]
                 + [pl.BlockSpec((1, 1, BR), m) for m in wmaps],
        out_specs=pl.BlockSpec((BR, 128), lambda i: (i, 0)),
        out_shape=jax.ShapeDtypeStruct((E4, 128), jnp.float32),
    )(rvt, rvt, rvt, rvt, w3d, w3d, w3d, w3d)


def _make_phase_a():
    mesh = plsc.VectorSubcoreMesh(core_axis_name="c", subcore_axis_name="s")

    @functools.partial(
        pl.kernel,
        out_type=jax.ShapeDtypeStruct((N_NODES, 32), jnp.float32),
        mesh=mesh,
        scratch_types=[
            pltpu.VMEM_SHARED((N_NODES, 16), jnp.float32),
            pltpu.VMEM((2, CHA, G), jnp.int32),
            pltpu.VMEM((3, CHA, G), jnp.int32),
            pltpu.VMEM((2, CHA, G), jnp.float32),
            pltpu.VMEM((3, CHA, G, 16), jnp.float32),
            pltpu.VMEM((ZROWS, 16), jnp.float32),
            pltpu.SemaphoreType.DMA,
            pltpu.SemaphoreType.DMA,
            pltpu.SemaphoreType.DMA((3,)),
        ],
        compiler_params=pltpu.CompilerParams(use_tc_tiling_on_sc=False),
    )
    def ka(*refs):
        _sc_body_a(refs)

    return ka


def _make_phase_b():
    mesh = plsc.VectorSubcoreMesh(core_axis_name="c", subcore_axis_name="s")

    @functools.partial(
        pl.kernel,
        out_type=jax.ShapeDtypeStruct((N_NODES, 32), jnp.float32),
        mesh=mesh,
        scratch_types=[
            pltpu.VMEM_SHARED((N_NODES, 16), jnp.float32),
            pltpu.VMEM((2, 4, G), jnp.int32),
            pltpu.VMEM((2, 4, G, 16), jnp.float32),
            pltpu.VMEM((ZROWS, 16), jnp.float32),
            pltpu.SemaphoreType.DMA((2, 4)),
            pltpu.SemaphoreType.DMA((2, 4)),
            pltpu.SemaphoreType.DMA((2, 4)),
        ],
        compiler_params=pltpu.CompilerParams(use_tc_tiling_on_sc=False),
    )
    def kb(*refs):
        _sc_body_b(refs)

    return kb


def _tc_matmul(a, b, wn_t, wr_t):
    BR = 2000

    def body(a_ref, b_ref, wn_ref, wr_ref, o_ref):
        o_ref[...] = (
            jnp.dot(a_ref[...], wn_ref[...], preferred_element_type=jnp.float32)
            + jnp.dot(b_ref[...], wr_ref[...], preferred_element_type=jnp.float32))

    return pl.pallas_call(
        body,
        grid=(N_NODES // BR,),
        in_specs=[pl.BlockSpec((BR, 32), lambda i: (i, 0)),
                  pl.BlockSpec((BR, 32), lambda i: (i, 0)),
                  pl.BlockSpec((32, 32), lambda i: (0, 0)),
                  pl.BlockSpec((32, 32), lambda i: (0, 0))],
        out_specs=pl.BlockSpec((BR, 32), lambda i: (i, 0)),
        out_shape=jax.ShapeDtypeStruct((N_NODES, 32), jnp.float32),
    )(a, b, wn_t, wr_t)


def kernel(feat, edge_index, review_feat, edge_weight, W_node, W_review):
    ei = edge_index.astype(jnp.int32)
    src = ei[0]
    dst = ei[1]
    w = edge_weight.reshape(-1)
    feat0 = feat[:, :16]
    feat1 = feat[:, 16:]
    rw = _tc_repack(review_feat.T, w.reshape(N_EDGES // 3200, 1, 3200))
    a64 = _make_phase_a()(feat0, feat1, src, dst, w)
    b64 = _make_phase_b()(rw, dst)
    return _tc_matmul(a64, b64, W_node.T, W_review.T)
